# baseline (speedup 1.0000x reference)
import functools

import jax
import jax.numpy as jnp
from jax.experimental import pallas as pl
from jax.experimental.pallas import tpu as pltpu


def _pos_body(row_ref, col_ref, out_hbm, scratch, sem, *, H, W, B):
    n_dim, HW = scratch.shape
    e = n_dim // 2
    col_t = col_ref[:W, :].T
    row_t = row_ref[:H, :].T
    scratch[:e, :] = jnp.broadcast_to(col_t[:, None, :], (e, H, W)).reshape(e, HW)
    scratch[e:, :] = jnp.broadcast_to(row_t[:, :, None], (e, H, W)).reshape(e, HW)
    cp = pltpu.make_async_copy(scratch, out_hbm.at[0], sem)
    cp.start()
    cp.wait()
    k = 1
    while k < B:
        n = min(k, B - k)
        cp = pltpu.make_async_copy(
            out_hbm.at[pl.ds(0, n)], out_hbm.at[pl.ds(k, n)], sem
        )
        cp.start()
        cp.wait()
        k += k


def kernel(x, row_embed, col_embed):
    B = x.shape[0]
    H, W = x.shape[-2], x.shape[-1]
    e = row_embed.shape[1]
    n_dim = 2 * e
    out = pl.pallas_call(
        functools.partial(_pos_body, H=H, W=W, B=B),
        in_specs=[
            pl.BlockSpec(memory_space=pltpu.MemorySpace.VMEM),
            pl.BlockSpec(memory_space=pltpu.MemorySpace.VMEM),
        ],
        out_specs=pl.BlockSpec(memory_space=pltpu.MemorySpace.HBM),
        out_shape=jax.ShapeDtypeStruct((B, n_dim, H * W), row_embed.dtype),
        scratch_shapes=[
            pltpu.VMEM((n_dim, H * W), row_embed.dtype),
            pltpu.SemaphoreType.DMA,
        ],
    )(row_embed, col_embed)
    return out.reshape(B, n_dim, H, W)


# speedup vs baseline: 12.7231x; 12.7231x over previous
import functools

import jax
import jax.numpy as jnp
from jax import lax
from jax.experimental import pallas as pl
from jax.experimental.pallas import tpu as pltpu
from jax.experimental.pallas import tpu_sc as plsc


def _sc_body(tbl_hbm, out_hbm, tbl, pattern, in_sem, out_sem, *, B, e, H, W):
    c = lax.axis_index("c")
    s = lax.axis_index("s")
    t = c * 16 + s
    n_dim = 2 * e
    ch_per_tile = n_dim // 32
    ch0 = t * ch_per_tile

    cp = pltpu.make_async_copy(tbl_hbm, tbl, in_sem)
    cp.start()
    cp.wait()

    iota = lax.iota(jnp.int32, 16)
    is_col = t * ch_per_tile < e

    def seg(j, _):
        w_col = iota + (j % 2) * 16
        w_row = jnp.zeros((16,), jnp.int32) + j // 2
        w = jnp.where(is_col, w_col, w_row)
        base = w * n_dim + ch0
        for i in range(ch_per_tile):
            v = plsc.load_gather(tbl, [base + i])
            pattern[i, pl.ds(j * 16, 16)] = v
        return 0

    lax.fori_loop(0, (H * W) // 16, seg, 0)

    for b in range(B):
        pltpu.make_async_copy(
            pattern, out_hbm.at[b, pl.ds(ch0, ch_per_tile), :], out_sem
        ).start()
    for b in range(B):
        pltpu.make_async_copy(
            pattern, out_hbm.at[b, pl.ds(ch0, ch_per_tile), :], out_sem
        ).wait()


def kernel(x, row_embed, col_embed):
    B = x.shape[0]
    H, W = x.shape[-2], x.shape[-1]
    e = row_embed.shape[1]
    n_dim = 2 * e
    tbl = jnp.concatenate([col_embed, row_embed], axis=1).reshape(-1)
    body = functools.partial(_sc_body, B=B, e=e, H=H, W=W)
    out = pl.kernel(
        body,
        out_type=jax.ShapeDtypeStruct((B, n_dim, H * W), row_embed.dtype),
        mesh=plsc.VectorSubcoreMesh(core_axis_name="c", subcore_axis_name="s"),
        scratch_types=[
            pltpu.VMEM(tbl.shape, tbl.dtype),
            pltpu.VMEM((n_dim // 32, H * W), row_embed.dtype),
            pltpu.SemaphoreType.DMA,
            pltpu.SemaphoreType.DMA,
        ],
        compiler_params=pltpu.CompilerParams(needs_layout_passes=False),
    )(tbl)
    return out.reshape(B, n_dim, H, W)
